# baseline (speedup 1.0000x reference)
import functools

import jax
import jax.numpy as jnp
from jax import lax
from jax.experimental import pallas as pl
from jax.experimental.pallas import tpu as pltpu
from jax.experimental.pallas import tpu_sc as plsc

N = 10000
D_IN = 128
D_OUT = 128
BR = 200
NBLK = N // (2 * BR)
_LANES = 16

ROWS_MAIN = 312
ROWS_LAST = 328


def _half(blk, x, wt, b, h_ref, val_ref, idx_ref):
    maxv = jnp.max(blk, axis=1)
    iota = lax.broadcasted_iota(jnp.int32, (1, N), 1)
    cand = jnp.where(blk == maxv[:, None], iota, N)
    idx = jnp.min(cand, axis=1)
    val_ref[0, 0, :] = maxv
    idx_ref[0, 0, :] = idx
    h_ref[...] = jnp.dot(x, wt, preferred_element_type=jnp.float32) + b


def _tc_body(ppr1_ref, ppr2_ref, x1_ref, x2_ref, wt_ref, b_ref,
             h1_ref, h2_ref, v1_ref, v2_ref, i1_ref, i2_ref):
    wt = wt_ref[...]
    b = b_ref[...]
    _half(ppr1_ref[...], x1_ref[...], wt, b, h1_ref, v1_ref, i1_ref)
    _half(ppr2_ref[...], x2_ref[...], wt, b, h2_ref, v2_ref, i2_ref)


_tc_call = pl.pallas_call(
    _tc_body,
    grid=(NBLK,),
    in_specs=[
        pl.BlockSpec((BR, N), lambda i: (i, 0)),
        pl.BlockSpec((BR, N), lambda i: (i + NBLK, 0)),
        pl.BlockSpec((BR, D_IN), lambda i: (i, 0)),
        pl.BlockSpec((BR, D_IN), lambda i: (i + NBLK, 0)),
        pl.BlockSpec((D_IN, D_OUT), lambda i: (0, 0)),
        pl.BlockSpec((1, D_OUT), lambda i: (0, 0)),
    ],
    out_specs=[
        pl.BlockSpec((BR, D_OUT), lambda i: (i, 0)),
        pl.BlockSpec((BR, D_OUT), lambda i: (i, 0)),
        pl.BlockSpec((1, 1, BR), lambda i: (i, 0, 0)),
        pl.BlockSpec((1, 1, BR), lambda i: (i, 0, 0)),
        pl.BlockSpec((1, 1, BR), lambda i: (i, 0, 0)),
        pl.BlockSpec((1, 1, BR), lambda i: (i, 0, 0)),
    ],
    out_shape=[
        jax.ShapeDtypeStruct((N // 2, D_OUT), jnp.float32),
        jax.ShapeDtypeStruct((N // 2, D_OUT), jnp.float32),
        jax.ShapeDtypeStruct((NBLK, 1, BR), jnp.float32),
        jax.ShapeDtypeStruct((NBLK, 1, BR), jnp.float32),
        jax.ShapeDtypeStruct((NBLK, 1, BR), jnp.int32),
        jax.ShapeDtypeStruct((NBLK, 1, BR), jnp.int32),
    ],
)


@functools.cache
def _get_sc_gather_scale():
    @functools.partial(
        pl.kernel,
        out_type=jax.ShapeDtypeStruct((N, D_OUT), jnp.float32),
        mesh=plsc.VectorSubcoreMesh(core_axis_name="c", subcore_axis_name="s"),
        scratch_types=[
            pltpu.VMEM((ROWS_LAST,), jnp.int32),
            pltpu.VMEM((ROWS_LAST + _LANES,), jnp.float32),
            pltpu.VMEM((ROWS_LAST, D_OUT), jnp.float32),
            pltpu.SemaphoreType.DMA,
        ],
    )
    def _sc_gather_scale(h_hbm, idx_hbm, val_hbm, out_hbm, idx_v, val_v, rows_v, sem):
        wid = lax.axis_index("s") * 2 + lax.axis_index("c")
        base = wid * ROWS_MAIN
        nrows = jnp.where(wid == 31, ROWS_LAST, ROWS_MAIN)

        def stage(rows):
            pltpu.sync_copy(idx_hbm.at[pl.ds(base, rows)], idx_v.at[pl.ds(0, rows)])
            pltpu.sync_copy(val_hbm.at[pl.ds(base, rows)], val_v.at[pl.ds(0, rows)])
            pltpu.async_copy(
                h_hbm.at[idx_v.at[pl.ds(0, rows)]], rows_v.at[pl.ds(0, rows)], sem
            ).wait()

        @pl.when(wid == 31)
        def _():
            stage(ROWS_LAST)

        @pl.when(wid != 31)
        def _():
            stage(ROWS_MAIN)

        def body(r, carry):
            vchunk = val_v[pl.ds(r, _LANES)]
            w = jnp.full((_LANES,), vchunk[0], jnp.float32)
            for j in range(D_OUT // _LANES):
                sl = pl.ds(j * _LANES, _LANES)
                rows_v[r, sl] = rows_v[r, sl] * w
            return carry

        lax.fori_loop(0, nrows, body, 0)

        @pl.when(wid == 31)
        def _():
            pltpu.sync_copy(rows_v, out_hbm.at[pl.ds(base, ROWS_LAST)])

        @pl.when(wid != 31)
        def _():
            pltpu.sync_copy(
                rows_v.at[pl.ds(0, ROWS_MAIN)], out_hbm.at[pl.ds(base, ROWS_MAIN)]
            )

    return _sc_gather_scale


def kernel(x, ppr_info, W, b):
    wt = W.T
    b2 = b.reshape(1, D_OUT)
    h1, h2, v1, v2, i1, i2 = _tc_call(ppr_info, ppr_info, x, x, wt, b2)
    H = jnp.concatenate([h1, h2], axis=0)
    val = jnp.concatenate([v1.reshape(N // 2), v2.reshape(N // 2)])
    idx = jnp.concatenate([i1.reshape(N // 2), i2.reshape(N // 2)])
    return _get_sc_gather_scale()(H, idx, val)


# speedup vs baseline: 1.0468x; 1.0468x over previous
import functools

import jax
import jax.numpy as jnp
from jax import lax
from jax.experimental import pallas as pl
from jax.experimental.pallas import tpu as pltpu
from jax.experimental.pallas import tpu_sc as plsc

N = 10000
D_IN = 128
D_OUT = 128
BR = 400
NBLK = N // BR
_LANES = 16

ROWS_MAIN = 312
ROWS_LAST = 328


def _tc_body(ppr_ref, x_ref, wt_ref, b_ref, h_ref, val_ref, idx_ref):
    blk = ppr_ref[...]
    maxv = jnp.max(blk, axis=1)
    iota = lax.broadcasted_iota(jnp.int32, (1, N), 1)
    cand = jnp.where(blk == maxv[:, None], iota, N)
    idx = jnp.min(cand, axis=1)
    val_ref[0, 0, :] = maxv
    idx_ref[0, 0, :] = idx
    h_ref[...] = (
        jnp.dot(x_ref[...], wt_ref[...], preferred_element_type=jnp.float32)
        + b_ref[...]
    )


_tc_call = pl.pallas_call(
    _tc_body,
    grid=(NBLK,),
    in_specs=[
        pl.BlockSpec((BR, N), lambda i: (i, 0)),
        pl.BlockSpec((BR, D_IN), lambda i: (i, 0)),
        pl.BlockSpec((D_IN, D_OUT), lambda i: (0, 0)),
        pl.BlockSpec((1, D_OUT), lambda i: (0, 0)),
    ],
    out_specs=[
        pl.BlockSpec((BR, D_OUT), lambda i: (i, 0)),
        pl.BlockSpec((1, 1, BR), lambda i: (i, 0, 0)),
        pl.BlockSpec((1, 1, BR), lambda i: (i, 0, 0)),
    ],
    out_shape=[
        jax.ShapeDtypeStruct((N, D_OUT), jnp.float32),
        jax.ShapeDtypeStruct((NBLK, 1, BR), jnp.float32),
        jax.ShapeDtypeStruct((NBLK, 1, BR), jnp.int32),
    ],
)


@functools.cache
def _get_sc_gather_scale():
    @functools.partial(
        pl.kernel,
        out_type=jax.ShapeDtypeStruct((N, D_OUT), jnp.float32),
        mesh=plsc.VectorSubcoreMesh(core_axis_name="c", subcore_axis_name="s"),
        scratch_types=[
            pltpu.VMEM((ROWS_LAST,), jnp.int32),
            pltpu.VMEM((ROWS_LAST + _LANES,), jnp.float32),
            pltpu.VMEM((ROWS_LAST, D_OUT), jnp.float32),
            pltpu.SemaphoreType.DMA,
        ],
    )
    def _sc_gather_scale(h_hbm, idx_hbm, val_hbm, out_hbm, idx_v, val_v, rows_v, sem):
        wid = lax.axis_index("s") * 2 + lax.axis_index("c")
        base = wid * ROWS_MAIN
        nrows = jnp.where(wid == 31, ROWS_LAST, ROWS_MAIN)

        def stage(rows):
            pltpu.sync_copy(idx_hbm.at[pl.ds(base, rows)], idx_v.at[pl.ds(0, rows)])
            pltpu.sync_copy(val_hbm.at[pl.ds(base, rows)], val_v.at[pl.ds(0, rows)])
            pltpu.async_copy(
                h_hbm.at[idx_v.at[pl.ds(0, rows)]], rows_v.at[pl.ds(0, rows)], sem
            ).wait()

        @pl.when(wid == 31)
        def _():
            stage(ROWS_LAST)

        @pl.when(wid != 31)
        def _():
            stage(ROWS_MAIN)

        def body(r, carry):
            vchunk = val_v[pl.ds(r, _LANES)]
            w = jnp.full((_LANES,), vchunk[0], jnp.float32)
            for j in range(D_OUT // _LANES):
                sl = pl.ds(j * _LANES, _LANES)
                rows_v[r, sl] = rows_v[r, sl] * w
            return carry

        lax.fori_loop(0, nrows, body, 0)

        @pl.when(wid == 31)
        def _():
            pltpu.sync_copy(rows_v, out_hbm.at[pl.ds(base, ROWS_LAST)])

        @pl.when(wid != 31)
        def _():
            pltpu.sync_copy(
                rows_v.at[pl.ds(0, ROWS_MAIN)], out_hbm.at[pl.ds(base, ROWS_MAIN)]
            )

    return _sc_gather_scale


def kernel(x, ppr_info, W, b):
    wt = W.T
    b2 = b.reshape(1, D_OUT)
    H, val3, idx3 = _tc_call(ppr_info, x, wt, b2)
    val = val3.reshape(N)
    idx = idx3.reshape(N)
    return _get_sc_gather_scale()(H, idx, val)


# speedup vs baseline: 1.0546x; 1.0075x over previous
import functools

import jax
import jax.numpy as jnp
from jax import lax
from jax.experimental import pallas as pl
from jax.experimental.pallas import tpu as pltpu
from jax.experimental.pallas import tpu_sc as plsc

N = 10000
D_IN = 128
D_OUT = 128
BR = 400
NBLK = N // BR
_LANES = 16

ROWS_MAIN = 312
ROWS_LAST = 328


def _tc_body(ppr_ref, x_ref, wt_ref, b_ref, h_ref, val_ref, idx_ref):
    blk = ppr_ref[...]
    maxv = jnp.max(blk, axis=1)
    iota = lax.broadcasted_iota(jnp.int32, (1, N), 1)
    cand = jnp.where(blk == maxv[:, None], iota, N)
    idx = jnp.min(cand, axis=1)
    val_ref[0, 0, :] = maxv
    idx_ref[0, 0, :] = idx
    h_ref[...] = (
        jnp.dot(x_ref[...], wt_ref[...], preferred_element_type=jnp.float32)
        + b_ref[...]
    )


_tc_call = pl.pallas_call(
    _tc_body,
    grid=(NBLK,),
    in_specs=[
        pl.BlockSpec((BR, N), lambda i: (i, 0)),
        pl.BlockSpec((BR, D_IN), lambda i: (i, 0)),
        pl.BlockSpec((D_IN, D_OUT), lambda i: (0, 0)),
        pl.BlockSpec((1, D_OUT), lambda i: (0, 0)),
    ],
    out_specs=[
        pl.BlockSpec((BR, D_OUT), lambda i: (i, 0)),
        pl.BlockSpec((1, 1, BR), lambda i: (i, 0, 0)),
        pl.BlockSpec((1, 1, BR), lambda i: (i, 0, 0)),
    ],
    out_shape=[
        jax.ShapeDtypeStruct((N, D_OUT), jnp.float32),
        jax.ShapeDtypeStruct((NBLK, 1, BR), jnp.float32),
        jax.ShapeDtypeStruct((NBLK, 1, BR), jnp.int32),
    ],
)


@functools.cache
def _get_sc_gather_scale():
    @functools.partial(
        pl.kernel,
        out_type=jax.ShapeDtypeStruct((N, D_OUT), jnp.float32),
        mesh=plsc.VectorSubcoreMesh(core_axis_name="c", subcore_axis_name="s"),
        scratch_types=[
            pltpu.VMEM((ROWS_LAST,), jnp.int32),
            pltpu.VMEM((ROWS_LAST + _LANES,), jnp.float32),
            pltpu.VMEM((ROWS_LAST, D_OUT), jnp.float32),
            pltpu.SemaphoreType.DMA,
        ],
    )
    def _sc_gather_scale(h_hbm, idx_hbm, val_hbm, out_hbm, idx_v, val_v, rows_v, sem):
        wid = lax.axis_index("s") * 2 + lax.axis_index("c")
        base = wid * ROWS_MAIN
        nrows = jnp.where(wid == 31, ROWS_LAST, ROWS_MAIN)

        def stage(rows):
            pltpu.sync_copy(idx_hbm.at[pl.ds(base, rows)], idx_v.at[pl.ds(0, rows)])
            pltpu.sync_copy(val_hbm.at[pl.ds(base, rows)], val_v.at[pl.ds(0, rows)])
            pltpu.async_copy(
                h_hbm.at[idx_v.at[pl.ds(0, rows)]], rows_v.at[pl.ds(0, rows)], sem
            ).wait()

        @pl.when(wid == 31)
        def _():
            stage(ROWS_LAST)

        @pl.when(wid != 31)
        def _():
            stage(ROWS_MAIN)

        @plsc.parallel_loop(0, nrows, step=1, unroll=4)
        def _scale(r):
            vchunk = val_v[pl.ds(r, _LANES)]
            w = jnp.full((_LANES,), vchunk[0], jnp.float32)
            for j in range(D_OUT // _LANES):
                sl = pl.ds(j * _LANES, _LANES)
                rows_v[r, sl] = rows_v[r, sl] * w

        @pl.when(wid == 31)
        def _():
            pltpu.sync_copy(rows_v, out_hbm.at[pl.ds(base, ROWS_LAST)])

        @pl.when(wid != 31)
        def _():
            pltpu.sync_copy(
                rows_v.at[pl.ds(0, ROWS_MAIN)], out_hbm.at[pl.ds(base, ROWS_MAIN)]
            )

    return _sc_gather_scale


def kernel(x, ppr_info, W, b):
    wt = W.T
    b2 = b.reshape(1, D_OUT)
    H, val3, idx3 = _tc_call(ppr_info, x, wt, b2)
    val = val3.reshape(N)
    idx = idx3.reshape(N)
    return _get_sc_gather_scale()(H, idx, val)
